# baseline (speedup 1.0000x reference)
import jax
import jax.numpy as jnp
from jax import lax
from jax.experimental import pallas as pl
from jax.experimental.pallas import tpu as pltpu
from jax.experimental.pallas import tpu_sc as plsc

_VOCAB = 1000000
_D = 64
_DP = 128
_B = 16384
_S = 50
_SP = 56
_LANES = 16
_G = _D // _LANES

_NC, _NS = 2, 16
_NW = _NC * _NS

_IPS = 100
_RPC = 2
_OPC = _RPC * _SP
_ROWS_W = _B // _NW
_CH_W = _ROWS_W // _RPC
_NSTR = _B * _S // _IPS
_STR_W = _NSTR // _NW
_NBUF = 4
_K = 3
_NFLUSH = 8
_CH_F = _CH_W // _NFLUSH
_ROWS_F = _ROWS_W // _NFLUSH


def _body(idx_hbm, tab_hbm, out_hbm, avg_hbm,
          idx_v, rows_v, avg_v, gs0, gs1, gs2, gs3, os0, os1, os2, os3):
    gsems = (gs0, gs1, gs2, gs3)
    osems = (os0, os1, os2, os3)
    wid = lax.axis_index("s") * _NC + lax.axis_index("c")

    pltpu.sync_copy(idx_hbm.at[pl.ds(wid * _STR_W, _STR_W)], idx_v)

    def streams(b, c):
        for r in range(_RPC):
            yield (
                tab_hbm.at[idx_v.at[_RPC * c // 2 + r // 2,
                                    pl.ds((r % 2) * _S, _S)]],
                rows_v.at[b, pl.ds(r * _SP, _S)],
                gsems[b])

    def fire_gather(b, c):
        for src, dst, sem in streams(b, c):
            pltpu.async_copy(src, dst, sem)

    def wait_gather(b, c):
        for src, dst, sem in streams(b, c):
            pltpu.make_async_copy(src, dst, sem).wait()

    def out_slice(c):
        return out_hbm.at[pl.ds((wid * _CH_W + c) * _OPC, _OPC)]

    def fire_writeout(b, c):
        pltpu.async_copy(rows_v.at[b], out_slice(c), osems[b])

    def wait_writeout(b, c):
        pltpu.make_async_copy(rows_v.at[b], out_slice(c), osems[b]).wait()

    for c in range(_K):
        fire_gather(c % _NBUF, c)

    def wave(g, carry):
        for b in range(_NBUF):
            c = g * _NBUF + b
            wait_gather(b, c)
            fire_writeout(b, c)

            def sbody(s, acc):
                new = []
                for r in range(_RPC):
                    for gg in range(_G):
                        v = rows_v[b, r * _SP + s, pl.ds(gg * _LANES, _LANES)]
                        new.append(acc[r * _G + gg] + v)
                return tuple(new)

            acc0 = tuple(jnp.zeros((_LANES,), jnp.float32)
                         for _ in range(_RPC * _G))
            acc = lax.fori_loop(0, _S, sbody, acc0)
            inv = jnp.float32(1.0 / _S)
            vrow = (c % _CH_F) * _RPC
            for r in range(_RPC):
                for gg in range(_G):
                    avg_v[vrow + r, pl.ds(gg * _LANES, _LANES)] = (
                        acc[r * _G + gg] * inv)

            @pl.when(c % _CH_F == _CH_F - 1)
            def _():
                q = c // _CH_F
                pltpu.sync_copy(
                    avg_v,
                    avg_hbm.at[pl.ds(wid * _ROWS_W + q * _ROWS_F, _ROWS_F)])

            bp = (b + _K) % _NBUF
            cp = c + _K

            @pl.when(cp < _CH_W)
            def _():
                @pl.when(cp - _NBUF >= 0)
                def _():
                    wait_writeout(bp, cp - _NBUF)
                fire_gather(bp, cp)
        return carry

    lax.fori_loop(0, _CH_W // _NBUF, wave, 0)
    for c in range(_CH_W - _NBUF, _CH_W):
        wait_writeout(c % _NBUF, c)


_sc_call = pl.kernel(
    _body,
    out_type=(
        jax.ShapeDtypeStruct((_B * _SP, _DP), jnp.float32),
        jax.ShapeDtypeStruct((_B, _DP), jnp.float32),
    ),
    mesh=plsc.VectorSubcoreMesh(core_axis_name="c", subcore_axis_name="s"),
    scratch_types=[
        pltpu.VMEM((_STR_W, _DP), jnp.int32),
        pltpu.VMEM((_NBUF, _OPC, _DP), jnp.float32),
        pltpu.VMEM((_ROWS_F, _DP), jnp.float32),
    ] + [pltpu.SemaphoreType.DMA] * (2 * _NBUF),
)


def _avg_cols(avgp):
    return avgp[:, :_D]


@jax.jit
def kernel(inputs, mask, embed_weight):
    del mask
    idx2 = jnp.pad(inputs.reshape(_NSTR, _IPS).astype(jnp.int32),
                   ((0, 0), (0, _DP - _IPS)))
    tabp = jnp.pad(embed_weight, ((0, 0), (0, _DP - _D)))
    outp, avgp = _sc_call(idx2, tabp)
    out = outp.reshape(_B, _SP, _DP)[:, :_S, :_D]
    return out, _avg_cols(avgp)
